# baseline (speedup 1.0000x reference)
import functools

import jax
import jax.numpy as jnp
from jax import lax
from jax.experimental import pallas as pl
from jax.experimental.pallas import tpu as pltpu
from jax.experimental.pallas import tpu_sc as plsc

N = 4096
D = 2048
B = 512

NC = 2
NS = 16
NW = NC * NS
RPW = N // NW
BAND = RPW * N
ZB = 16384
NCOPY = BAND // ZB


def _tc_kernel(ub_ref, lb_ref, Wu_in_ref, bu_in_ref, Wl_in_ref, bl_in_ref,
               alpha_ref,
               new_ub_ref, new_lb_ref, Wu_ref, bu_ref, Wl_ref, bl_ref,
               bu2_ref, bl2_ref):
    ub = ub_ref[:]
    lb = lb_ref[:]
    alpha = alpha_ref[:]
    bu_in = bu_in_ref[:]
    bl_in = bl_in_ref[:]

    neg = ub <= 0.0
    pos = lb >= 0.0
    cross = jnp.logical_not(jnp.logical_or(neg, pos))
    alpha_c = jnp.clip(alpha, 0.0, 1.0)
    denom = jnp.where(cross, ub - lb, 1.0)
    a = jnp.where(cross, ub / denom, 0.0)
    b = -lb * a

    new_ub_ref[:] = jnp.where(neg, 0.0, ub)
    new_lb_ref[:] = jnp.where(pos, lb, jnp.where(cross, alpha_c * lb, 0.0))
    bu_ref[:] = jnp.where(pos, bu_in, jnp.where(cross, bu_in + b, 0.0))
    bl_ref[:] = jnp.where(pos, bl_in, jnp.where(cross, bu_in, 0.0))
    bu2_ref[:] = jnp.where(cross, b, 0.0)
    bl2_ref[:] = jnp.zeros_like(b)

    u_scale = jnp.where(pos, 1.0, a)
    l_scale = jnp.where(pos, 1.0, jnp.where(cross, alpha_c, 0.0))
    Wu_ref[:, :] = u_scale[:, None] * Wu_in_ref[:, :]
    Wl_ref[:, :] = l_scale[:, None] * Wl_in_ref[:, :]


def _tc_part(ub, lb, W_upper, b_upper, W_lower, b_lower, alpha):
    grid = (N // B,)
    vec_spec = pl.BlockSpec((B,), lambda i: (i,))
    mat_spec = pl.BlockSpec((B, D), lambda i: (i, 0))
    f32 = jnp.float32
    out_shapes = (
        jax.ShapeDtypeStruct((N,), f32),
        jax.ShapeDtypeStruct((N,), f32),
        jax.ShapeDtypeStruct((N, D), f32),
        jax.ShapeDtypeStruct((N,), f32),
        jax.ShapeDtypeStruct((N, D), f32),
        jax.ShapeDtypeStruct((N,), f32),
        jax.ShapeDtypeStruct((N,), f32),
        jax.ShapeDtypeStruct((N,), f32),
    )
    out_specs = (vec_spec, vec_spec, mat_spec, vec_spec, mat_spec, vec_spec,
                 vec_spec, vec_spec)
    in_specs = (vec_spec, vec_spec, mat_spec, vec_spec, mat_spec, vec_spec,
                vec_spec)
    return pl.pallas_call(
        _tc_kernel,
        grid=grid,
        in_specs=in_specs,
        out_specs=out_specs,
        out_shape=out_shapes,
    )(ub, lb, W_upper, b_upper, W_lower, b_lower, alpha)


def _sc_body(ub_hbm, lb_hbm, alpha_hbm, wu2_hbm, wl2_hbm,
             zb, ub_v, lb_v, al_v, vals_u, vals_l, idx_v, sem, sem2):
    wid = lax.axis_index("s") * NC + lax.axis_index("c")
    base = wid * BAND

    def _zero(i, _):
        zb[pl.ds(i * 16, 16)] = jnp.zeros((16,), jnp.float32)
        return 0
    lax.fori_loop(0, ZB // 16, _zero, 0)

    pltpu.sync_copy(ub_hbm.at[pl.ds(wid * RPW, RPW)], ub_v)
    pltpu.sync_copy(lb_hbm.at[pl.ds(wid * RPW, RPW)], lb_v)
    pltpu.sync_copy(alpha_hbm.at[pl.ds(wid * RPW, RPW)], al_v)

    copies = []
    for c in range(NCOPY):
        copies.append(pltpu.async_copy(
            zb, wu2_hbm.at[pl.ds(base + c * ZB, ZB)], sem))
        copies.append(pltpu.async_copy(
            zb, wl2_hbm.at[pl.ds(base + c * ZB, ZB)], sem))

    for j in range(RPW // 16):
        u = ub_v[pl.ds(j * 16, 16)]
        l = lb_v[pl.ds(j * 16, 16)]
        al = al_v[pl.ds(j * 16, 16)]
        cf = jnp.where(u > 0.0, 1.0, 0.0) * jnp.where(l < 0.0, 1.0, 0.0)
        alc = jnp.minimum(jnp.maximum(al, 0.0), 1.0)
        denom = cf * (u - l) + (1.0 - cf)
        a = cf * (u / denom)
        vals_u[pl.ds(j * 16, 16)] = cf * a + (1.0 - cf)
        vals_l[pl.ds(j * 16, 16)] = cf * alc + (1.0 - cf)
        rows = lax.broadcasted_iota(jnp.int32, (16,), 0) + (wid * RPW + j * 16)
        idx_v[pl.ds(j * 16, 16)] = rows * (N + 1)

    for cp in copies:
        cp.wait()

    pltpu.async_copy(vals_u, wu2_hbm.at[idx_v], sem2).wait()
    pltpu.async_copy(vals_l, wl2_hbm.at[idx_v], sem2).wait()


def _sc_diag(ub, lb, alpha):
    f32 = jnp.float32
    mesh = plsc.VectorSubcoreMesh(core_axis_name="c", subcore_axis_name="s")
    diag_kernel = pl.kernel(
        _sc_body,
        out_type=(
            jax.ShapeDtypeStruct((N * N,), f32),
            jax.ShapeDtypeStruct((N * N,), f32),
        ),
        mesh=mesh,
        scratch_types=(
            pltpu.VMEM((ZB,), f32),
            pltpu.VMEM((RPW,), f32),
            pltpu.VMEM((RPW,), f32),
            pltpu.VMEM((RPW,), f32),
            pltpu.VMEM((RPW,), f32),
            pltpu.VMEM((RPW,), f32),
            pltpu.VMEM((RPW,), jnp.int32),
            pltpu.SemaphoreType.DMA,
            pltpu.SemaphoreType.DMA,
        ),
    )
    return diag_kernel(ub, lb, alpha)


@jax.jit
def kernel(ub, lb, W_upper, b_upper, W_lower, b_lower, alpha, input_ub, input_lb):
    del input_ub, input_lb
    wu2_flat, wl2_flat = _sc_diag(ub, lb, alpha)
    (new_ub, new_lb, Wu, bu, Wl, bl, bu2, bl2) = _tc_part(
        ub, lb, W_upper, b_upper, W_lower, b_lower, alpha)
    return (new_ub, new_lb, Wu, bu, Wl, bl,
            wu2_flat.reshape(N, N), bu2, wl2_flat.reshape(N, N), bl2)


# speedup vs baseline: 2.5134x; 2.5134x over previous
import jax
import jax.numpy as jnp
from jax import lax
from jax.experimental import pallas as pl
from jax.experimental.pallas import tpu as pltpu
from jax.experimental.pallas import tpu_sc as plsc

N = 4096
D = 2048
B = 512

NC = 2
NS = 16
NW = NC * NS
RPW = N // NW
RC = 8
NCHUNK = RPW // RC


def _tc_kernel(ub_ref, lb_ref, Wu_in_ref, bu_in_ref, Wl_in_ref, bl_in_ref,
               alpha_ref,
               new_ub_ref, new_lb_ref, Wu_ref, bu_ref, Wl_ref, bl_ref,
               Wu2_ref, bu2_ref, bl2_ref):
    i = pl.program_id(0)
    ub = ub_ref[:]
    lb = lb_ref[:]
    alpha = alpha_ref[:]
    bu_in = bu_in_ref[:]
    bl_in = bl_in_ref[:]

    neg = ub <= 0.0
    pos = lb >= 0.0
    cross = jnp.logical_not(jnp.logical_or(neg, pos))
    alpha_c = jnp.clip(alpha, 0.0, 1.0)
    denom = jnp.where(cross, ub - lb, 1.0)
    a = jnp.where(cross, ub / denom, 0.0)
    b = -lb * a

    new_ub_ref[:] = jnp.where(neg, 0.0, ub)
    new_lb_ref[:] = jnp.where(pos, lb, jnp.where(cross, alpha_c * lb, 0.0))
    bu_ref[:] = jnp.where(pos, bu_in, jnp.where(cross, bu_in + b, 0.0))
    bl_ref[:] = jnp.where(pos, bl_in, jnp.where(cross, bu_in, 0.0))
    bu2_ref[:] = jnp.where(cross, b, 0.0)
    bl2_ref[:] = jnp.zeros_like(b)

    u_scale = jnp.where(pos, 1.0, a)
    l_scale = jnp.where(pos, 1.0, jnp.where(cross, alpha_c, 0.0))
    Wu_ref[:, :] = u_scale[:, None] * Wu_in_ref[:, :]
    Wl_ref[:, :] = l_scale[:, None] * Wl_in_ref[:, :]

    du = jnp.where(cross, a, 1.0)
    rows = lax.broadcasted_iota(jnp.int32, (B, N), 0) + i * B
    cols = lax.broadcasted_iota(jnp.int32, (B, N), 1)
    Wu2_ref[:, :] = jnp.where(rows == cols, du[:, None], 0.0)


def _tc_part(ub, lb, W_upper, b_upper, W_lower, b_lower, alpha):
    grid = (N // B,)
    vec_spec = pl.BlockSpec((B,), lambda i: (i,))
    mat_spec = pl.BlockSpec((B, D), lambda i: (i, 0))
    diag_spec = pl.BlockSpec((B, N), lambda i: (i, 0))
    f32 = jnp.float32
    out_shapes = (
        jax.ShapeDtypeStruct((N,), f32),
        jax.ShapeDtypeStruct((N,), f32),
        jax.ShapeDtypeStruct((N, D), f32),
        jax.ShapeDtypeStruct((N,), f32),
        jax.ShapeDtypeStruct((N, D), f32),
        jax.ShapeDtypeStruct((N,), f32),
        jax.ShapeDtypeStruct((N, N), f32),
        jax.ShapeDtypeStruct((N,), f32),
        jax.ShapeDtypeStruct((N,), f32),
    )
    out_specs = (vec_spec, vec_spec, mat_spec, vec_spec, mat_spec, vec_spec,
                 diag_spec, vec_spec, vec_spec)
    in_specs = (vec_spec, vec_spec, mat_spec, vec_spec, mat_spec, vec_spec,
                vec_spec)
    return pl.pallas_call(
        _tc_kernel,
        grid=grid,
        in_specs=in_specs,
        out_specs=out_specs,
        out_shape=out_shapes,
    )(ub, lb, W_upper, b_upper, W_lower, b_lower, alpha)


def _sc_body(ub_hbm, lb_hbm, alpha_hbm, wl2_hbm,
             buf_a, buf_b, ub_v, lb_v, al_v, vals, sem):
    wid = lax.axis_index("s") * NC + lax.axis_index("c")
    row0 = wid * RPW

    pltpu.sync_copy(ub_hbm.at[pl.ds(row0, RPW)], ub_v)
    pltpu.sync_copy(lb_hbm.at[pl.ds(row0, RPW)], lb_v)
    pltpu.sync_copy(alpha_hbm.at[pl.ds(row0, RPW)], al_v)

    for j in range(RPW // 16):
        u = ub_v[pl.ds(j * 16, 16)]
        l = lb_v[pl.ds(j * 16, 16)]
        al = al_v[pl.ds(j * 16, 16)]
        cf = jnp.where(u > 0.0, 1.0, 0.0) * jnp.where(l < 0.0, 1.0, 0.0)
        alc = jnp.minimum(jnp.maximum(al, 0.0), 1.0)
        vals[pl.ds(j * 16, 16)] = cf * alc + (1.0 - cf)

    zeros16 = jnp.zeros((16,), jnp.float32)
    for buf in (buf_a, buf_b):
        for r in range(RC):
            def _zero(k, _, buf=buf, r=r):
                buf[r, pl.ds(k * 16, 16)] = zeros16
                return 0
            lax.fori_loop(0, N // 16, _zero, 0)

    lanes = lax.broadcasted_iota(jnp.int32, (16,), 0)
    rows_s = lanes % RC

    bufs = (buf_a, buf_b)
    copies = [None, None]
    dirty = [None, None]
    for c in range(NCHUNK):
        s = c % 2
        buf = bufs[s]
        if copies[s] is not None:
            copies[s].wait()
            plsc.store_scatter(buf, [rows_s, dirty[s]], zeros16)
        cols_s = (row0 + c * RC + lanes) % N
        v = vals[pl.ds(c * RC, 16)]
        v = jnp.where(lanes < RC, v, 0.0)
        plsc.store_scatter(buf, [rows_s, cols_s], v)
        dirty[s] = cols_s
        copies[s] = pltpu.async_copy(
            buf, wl2_hbm.at[pl.ds(row0 + c * RC, RC), :], sem)
    copies[0].wait()
    copies[1].wait()


def _sc_diag_lower(ub, lb, alpha):
    f32 = jnp.float32
    mesh = plsc.VectorSubcoreMesh(core_axis_name="c", subcore_axis_name="s")
    diag_kernel = pl.kernel(
        _sc_body,
        out_type=jax.ShapeDtypeStruct((N, N), f32),
        mesh=mesh,
        compiler_params=pltpu.CompilerParams(needs_layout_passes=False),
        scratch_types=(
            pltpu.VMEM((RC, N), f32),
            pltpu.VMEM((RC, N), f32),
            pltpu.VMEM((RPW,), f32),
            pltpu.VMEM((RPW,), f32),
            pltpu.VMEM((RPW,), f32),
            pltpu.VMEM((RPW + 16,), f32),
            pltpu.SemaphoreType.DMA,
        ),
    )
    return diag_kernel(ub, lb, alpha)


@jax.jit
def kernel(ub, lb, W_upper, b_upper, W_lower, b_lower, alpha, input_ub, input_lb):
    del input_ub, input_lb
    wl2 = _sc_diag_lower(ub, lb, alpha)
    (new_ub, new_lb, Wu, bu, Wl, bl, Wu2, bu2, bl2) = _tc_part(
        ub, lb, W_upper, b_upper, W_lower, b_lower, alpha)
    return (new_ub, new_lb, Wu, bu, Wl, bl, Wu2, bu2, wl2, bl2)
